# baseline (speedup 1.0000x reference)
import functools
import math

import jax
import jax.numpy as jnp
from jax import lax
from jax.experimental import pallas as pl
from jax.experimental.pallas import tpu as pltpu
from jax.experimental.pallas import tpu_sc as plsc

D = 64
SCALE = math.sqrt(D)
V = 1000000

NC = 2
NS = 16
NW = NC * NS

X_ROWS = 4096
X_COLS = 200

TB = 160
T_BLOCKS = V // TB
T_STEPS = 196


def _transpose_body(tt_hbm, trm_hbm, tb0, tb1, to0, to1,
                    is0, is1, os0, os1):
    tbufs = [tb0, tb1]
    tobufs = [to0, to1]
    in_sems = [is0, is1]
    out_sems = [os0, os1]

    wid = lax.axis_index("s") * NC + lax.axis_index("c")

    def block_of(step):
        t = wid + NW * step
        return jnp.where(t >= T_BLOCKS, t - T_BLOCKS, t)

    def fire_in(step, b):
        col0 = block_of(step) * TB
        pltpu.async_copy(tt_hbm.at[:, pl.ds(col0, TB)],
                         tbufs[b], in_sems[b])

    def wait_in(b):
        pltpu.make_async_copy(tt_hbm.at[:, pl.ds(0, TB)], tbufs[b],
                              in_sems[b]).wait()

    def transpose(b):
        tbuf, tobuf = tbufs[b], tobufs[b]

        @plsc.parallel_loop(0, TB, 1, unroll=8)
        def _(r):
            col = jnp.full((16,), r, jnp.int32)
            for d0 in range(0, D, 16):
                rows = d0 + lax.iota(jnp.int32, 16)
                tobuf[r, pl.ds(d0, 16)] = plsc.load_gather(tbuf, [rows, col])

    def fire_out(step, b):
        row0 = block_of(step) * TB
        pltpu.async_copy(tobufs[b], trm_hbm.at[pl.ds(row0, TB)],
                         out_sems[b])

    def wait_out(b):
        pltpu.make_async_copy(tobufs[b], trm_hbm.at[pl.ds(0, TB)],
                              out_sems[b]).wait()

    fire_in(0, 0)
    fire_in(1, 1)

    for b in range(2):
        wait_in(b)
        transpose(b)
        fire_out(b, b)
        fire_in(b + 2, b)

    def outer(g, _):
        for b in range(2):
            step = 2 * g + b
            wait_in(b)
            wait_out(b)
            transpose(b)
            fire_out(step, b)
            fire_in(step + 2, b)
        return 0

    lax.fori_loop(1, T_STEPS // 2 - 1, outer, 0)

    for b in range(2):
        step = T_STEPS - 2 + b
        wait_in(b)
        wait_out(b)
        transpose(b)
        fire_out(step, b)

    wait_out(0)
    wait_out(1)


def _transpose(table_t):
    mesh = plsc.VectorSubcoreMesh(core_axis_name="c", subcore_axis_name="s")
    k = functools.partial(
        pl.kernel,
        mesh=mesh,
        out_type=jax.ShapeDtypeStruct((V, D), jnp.float32),
        scratch_types=(
            [pltpu.VMEM((D, TB), jnp.float32)] * 2
            + [pltpu.VMEM((TB, D), jnp.float32)] * 2
            + [pltpu.SemaphoreType.DMA] * 4
        ),
        compiler_params=pltpu.CompilerParams(use_tc_tiling_on_sc=False, needs_layout_passes=False),
    )(_transpose_body)
    return k(table_t)


GCOLS_PER_W = X_ROWS // NW
NBUF_G = 4
NBUF_O = 2
G_STEPS = X_COLS


def _gather_body(xt_hbm, trm_hbm, out_hbm, idx_v,
                 g0, g1, g2, g3, o0, o1,
                 in_s0, in_s1, in_s2, in_s3, out_s0, out_s1):
    gbufs = [g0, g1, g2, g3]
    obufs = [o0, o1]
    in_sems = [in_s0, in_s1, in_s2, in_s3]
    out_sems = [out_s0, out_s1]

    wid = lax.axis_index("s") * NC + lax.axis_index("c")
    i0w = wid * GCOLS_PER_W

    pltpu.sync_copy(xt_hbm.at[:, pl.ds(i0w, GCOLS_PER_W)], idx_v)

    def fire_gather(j, b):
        pltpu.async_copy(trm_hbm.at[idx_v.at[j]], gbufs[b], in_sems[b])

    def wait_gather(b):
        pltpu.make_async_copy(trm_hbm.at[pl.ds(0, GCOLS_PER_W)], gbufs[b],
                              in_sems[b]).wait()

    def tr_scale(b, ob):
        gbuf, obuf = gbufs[b], obufs[ob]

        @plsc.parallel_loop(0, D, 1, unroll=2)
        def _(d):
            col = jnp.full((16,), d, jnp.int32)
            for i0 in range(0, GCOLS_PER_W, 16):
                rows = i0 + lax.iota(jnp.int32, 16)
                obuf[d, pl.ds(i0, 16)] = (
                    plsc.load_gather(gbuf, [rows, col]) * SCALE)

    def fire_write(j, ob):
        pltpu.async_copy(obufs[ob],
                         out_hbm.at[j, :, pl.ds(i0w, GCOLS_PER_W)],
                         out_sems[ob])

    def wait_write(ob):
        pltpu.make_async_copy(obufs[ob],
                              out_hbm.at[0, :, pl.ds(0, GCOLS_PER_W)],
                              out_sems[ob]).wait()

    for b in range(NBUF_G):
        fire_gather(b, b)

    for b in range(NBUF_G):
        wait_gather(b)
        if b >= NBUF_O:
            wait_write(b % NBUF_O)
        tr_scale(b, b % NBUF_O)
        fire_write(b, b % NBUF_O)
        fire_gather(b + NBUF_G, b)

    def outer(g, _):
        for b in range(NBUF_G):
            j = g * NBUF_G + b
            wait_gather(b)
            wait_write(b % NBUF_O)
            tr_scale(b, b % NBUF_O)
            fire_write(j, b % NBUF_O)
            fire_gather(j + NBUF_G, b)
        return 0

    lax.fori_loop(1, G_STEPS // NBUF_G - 1, outer, 0)

    for b in range(NBUF_G):
        j = G_STEPS - NBUF_G + b
        wait_gather(b)
        wait_write(b % NBUF_O)
        tr_scale(b, b % NBUF_O)
        fire_write(j, b % NBUF_O)

    for ob in range(NBUF_O):
        wait_write(ob)


def _gather(x_t, trm):
    mesh = plsc.VectorSubcoreMesh(core_axis_name="c", subcore_axis_name="s")
    k = functools.partial(
        pl.kernel,
        mesh=mesh,
        out_type=jax.ShapeDtypeStruct((X_COLS, D, X_ROWS), jnp.float32),
        scratch_types=(
            [pltpu.VMEM((X_COLS, GCOLS_PER_W), jnp.int32)]
            + [pltpu.VMEM((GCOLS_PER_W, D), jnp.float32)] * NBUF_G
            + [pltpu.VMEM((D, GCOLS_PER_W), jnp.float32)] * NBUF_O
            + [pltpu.SemaphoreType.DMA] * (NBUF_G + NBUF_O)
        ),
        compiler_params=pltpu.CompilerParams(use_tc_tiling_on_sc=False, needs_layout_passes=False),
    )(_gather_body)
    return k(x_t, trm)


def kernel(x, table):
    x_t = x.astype(jnp.int32).T
    table_t = table.T
    trm = _transpose(table_t)
    out_t = _gather(x_t, trm)
    return out_t.transpose(2, 0, 1)


# speedup vs baseline: 5.5757x; 5.5757x over previous
import functools
import math

import jax
import jax.numpy as jnp
from jax import lax
from jax.experimental import pallas as pl
from jax.experimental.pallas import tpu as pltpu
from jax.experimental.pallas import tpu_sc as plsc

D = 64
DP = 128
SCALE = math.sqrt(D)
V = 1000000

NC = 2
NS = 16
NW = NC * NS

X_ROWS = 4096
X_COLS = 200
GW = 128
NBUF_G = 4
NBUF_O = 2
G_STEPS = X_COLS


def _gather_body(xt_hbm, tp_hbm, out_hbm, idx_v,
                 g0, g1, g2, g3, o0, o1,
                 in_s0, in_s1, in_s2, in_s3, out_s0, out_s1):
    gbufs = [g0, g1, g2, g3]
    obufs = [o0, o1]
    in_sems = [in_s0, in_s1, in_s2, in_s3]
    out_sems = [out_s0, out_s1]

    wid = lax.axis_index("s") * NC + lax.axis_index("c")
    i0w = wid * GW

    pltpu.sync_copy(xt_hbm.at[:, pl.ds(i0w, GW)], idx_v)

    def fire_gather(j, b):
        pltpu.async_copy(tp_hbm.at[idx_v.at[j]], gbufs[b], in_sems[b])

    def wait_gather(b):
        pltpu.make_async_copy(tp_hbm.at[pl.ds(0, GW)], gbufs[b],
                              in_sems[b]).wait()

    def tr_scale(b, ob):
        gbuf, obuf = gbufs[b], obufs[ob]

        @plsc.parallel_loop(0, D, 1, unroll=2)
        def _(d):
            tr = d // 8
            r = d % 8
            col = jnp.full((16,), d, jnp.int32)
            for i0 in range(0, GW, 16):
                rows = i0 + lax.iota(jnp.int32, 16)
                obuf[tr, r, pl.ds(i0, 16)] = (
                    plsc.load_gather(gbuf, [rows, col]) * SCALE)

    def fire_write(j, ob):
        pltpu.async_copy(obufs[ob], out_hbm.at[j, :, wid, :, :],
                         out_sems[ob])

    def wait_write(ob):
        pltpu.make_async_copy(obufs[ob], out_hbm.at[0, :, 0, :, :],
                              out_sems[ob]).wait()

    for b in range(NBUF_G):
        fire_gather(b, b)

    for b in range(NBUF_G):
        wait_gather(b)
        if b >= NBUF_O:
            wait_write(b % NBUF_O)
        tr_scale(b, b % NBUF_O)
        fire_write(b, b % NBUF_O)
        fire_gather(b + NBUF_G, b)

    def outer(g, _):
        for b in range(NBUF_G):
            j = g * NBUF_G + b
            wait_gather(b)
            wait_write(b % NBUF_O)
            tr_scale(b, b % NBUF_O)
            fire_write(j, b % NBUF_O)
            fire_gather(j + NBUF_G, b)
        return 0

    lax.fori_loop(1, G_STEPS // NBUF_G - 1, outer, 0)

    for b in range(NBUF_G):
        j = (G_STEPS // NBUF_G - 1) * NBUF_G + b
        wait_gather(b)
        wait_write(b % NBUF_O)
        tr_scale(b, b % NBUF_O)
        fire_write(j, b % NBUF_O)

    for ob in range(NBUF_O):
        wait_write(ob)


def _gather(x_t, tpad):
    mesh = plsc.VectorSubcoreMesh(core_axis_name="c", subcore_axis_name="s")
    k = functools.partial(
        pl.kernel,
        mesh=mesh,
        out_type=jax.ShapeDtypeStruct((X_COLS, 8, NW, 8, 128), jnp.float32),
        scratch_types=(
            [pltpu.VMEM((X_COLS, GW), jnp.int32)]
            + [pltpu.VMEM((GW, DP), jnp.float32)] * NBUF_G
            + [pltpu.VMEM((8, 8, 128), jnp.float32)] * NBUF_O
            + [pltpu.SemaphoreType.DMA] * (NBUF_G + NBUF_O)
        ),
        compiler_params=pltpu.CompilerParams(use_tc_tiling_on_sc=False,
                                             needs_layout_passes=False),
    )(_gather_body)
    return k(x_t, tpad)


def kernel(x, table):
    x_t = x.astype(jnp.int32).T
    tpad = jnp.pad(table, ((0, 0), (0, DP - D)))
    out5 = _gather(x_t, tpad)
    out = out5.transpose(0, 1, 3, 2, 4).reshape(X_COLS, D, X_ROWS)
    return out.transpose(2, 0, 1)


# speedup vs baseline: 5.5769x; 1.0002x over previous
import functools
import math

import jax
import jax.numpy as jnp
from jax import lax
from jax.experimental import pallas as pl
from jax.experimental.pallas import tpu as pltpu
from jax.experimental.pallas import tpu_sc as plsc

D = 64
DP = 128
SCALE = math.sqrt(D)
V = 1000000

NC = 2
NS = 16
NW = NC * NS

X_ROWS = 4096
X_COLS = 200
GW = 128
NBUF_G = 4
NBUF_O = 4
G_STEPS = X_COLS


def _gather_body(xt_hbm, tp_hbm, out_hbm, idx_v,
                 g0, g1, g2, g3, o0, o1, o2, o3,
                 in_s0, in_s1, in_s2, in_s3,
                 out_s0, out_s1, out_s2, out_s3):
    gbufs = [g0, g1, g2, g3]
    obufs = [o0, o1, o2, o3]
    in_sems = [in_s0, in_s1, in_s2, in_s3]
    out_sems = [out_s0, out_s1, out_s2, out_s3]

    wid = lax.axis_index("s") * NC + lax.axis_index("c")
    i0w = wid * GW

    pltpu.sync_copy(xt_hbm.at[:, pl.ds(i0w, GW)], idx_v)

    def fire_gather(j, b):
        pltpu.async_copy(tp_hbm.at[idx_v.at[j]], gbufs[b], in_sems[b])

    def wait_gather(b):
        pltpu.make_async_copy(tp_hbm.at[pl.ds(0, GW)], gbufs[b],
                              in_sems[b]).wait()

    row_vecs = [i0 + lax.iota(jnp.int32, 16) for i0 in range(0, GW, 16)]

    def tr_scale(b, ob):
        gbuf, obuf = gbufs[b], obufs[ob]

        @plsc.parallel_loop(0, D, 1, unroll=4)
        def _(d):
            tr = lax.shift_right_logical(d, 3)
            r = lax.bitwise_and(d, 7)
            col = jnp.full((16,), d, jnp.int32)
            for k in range(GW // 16):
                obuf[tr, r, pl.ds(k * 16, 16)] = (
                    plsc.load_gather(gbuf, [row_vecs[k], col]) * SCALE)

    def fire_write(j, ob):
        pltpu.async_copy(obufs[ob], out_hbm.at[j, :, wid, :, :],
                         out_sems[ob])

    def wait_write(ob):
        pltpu.make_async_copy(obufs[ob], out_hbm.at[0, :, 0, :, :],
                              out_sems[ob]).wait()

    for b in range(NBUF_G):
        fire_gather(b, b)

    for b in range(NBUF_G):
        wait_gather(b)
        tr_scale(b, b % NBUF_O)
        fire_write(b, b % NBUF_O)
        fire_gather(b + NBUF_G, b)

    def outer(g, _):
        for b in range(NBUF_G):
            j = g * NBUF_G + b
            wait_gather(b)
            wait_write(b % NBUF_O)
            tr_scale(b, b % NBUF_O)
            fire_write(j, b % NBUF_O)
            fire_gather(j + NBUF_G, b)
        return 0

    lax.fori_loop(1, G_STEPS // NBUF_G - 1, outer, 0)

    for b in range(NBUF_G):
        j = (G_STEPS // NBUF_G - 1) * NBUF_G + b
        wait_gather(b)
        wait_write(b % NBUF_O)
        tr_scale(b, b % NBUF_O)
        fire_write(j, b % NBUF_O)

    for ob in range(NBUF_O):
        wait_write(ob)


def _gather(x_t, tpad):
    mesh = plsc.VectorSubcoreMesh(core_axis_name="c", subcore_axis_name="s")
    k = functools.partial(
        pl.kernel,
        mesh=mesh,
        out_type=jax.ShapeDtypeStruct((X_COLS, 8, NW, 8, 128), jnp.float32),
        scratch_types=(
            [pltpu.VMEM((X_COLS, GW), jnp.int32)]
            + [pltpu.VMEM((GW, DP), jnp.float32)] * NBUF_G
            + [pltpu.VMEM((8, 8, 128), jnp.float32)] * NBUF_O
            + [pltpu.SemaphoreType.DMA] * (NBUF_G + NBUF_O)
        ),
        compiler_params=pltpu.CompilerParams(use_tc_tiling_on_sc=False,
                                             needs_layout_passes=False),
    )(_gather_body)
    return k(x_t, tpad)


def kernel(x, table):
    x_t = x.astype(jnp.int32).T
    tpad = jnp.pad(table, ((0, 0), (0, DP - D)))
    out5 = _gather(x_t, tpad)
    out = out5.transpose(0, 1, 3, 2, 4).reshape(X_COLS, D, X_ROWS)
    return out.transpose(2, 0, 1)
